# Initial kernel scaffold; baseline (speedup 1.0000x reference)
import jax
import jax.numpy as jnp
from jax.experimental import pallas as pl


def kernel(x, edge_index, batch, y, p, c, apf, wiener, W1, b1, W2, b2, W3, b3, W4, b4):
    raise NotImplementedError("write your pallas kernel here")


# speedup vs baseline: 113.4888x; 113.4888x over previous
import functools

import jax
import jax.numpy as jnp
from jax import lax
from jax.experimental import pallas as pl
from jax.experimental.pallas import tpu as pltpu
from jax.experimental.pallas import tpu_sc as plsc

NN = 100000
EE = 3200000
BB = 1024
NPAD = 100352
RROWS = NPAD // 128
RPT = NPAD // 16
ER = EE // 128
MAIN_MACROS = 97
EPT_MAIN = MAIN_MACROS * 1024
MAIN_ROWS = EPT_MAIN // 128
LEFT_ROW0 = 32 * MAIN_ROWS
LEFT_MACROS = 21
SUPERS = 12
BPOOL = BB + 128

_MESH = plsc.VectorSubcoreMesh(
    core_axis_name="c", subcore_axis_name="s", num_cores=2, num_subcores=16)
_SC_PARAMS = pltpu.CompilerParams(use_tc_tiling_on_sc=False)

_f32 = jnp.float32
_i32 = jnp.int32


@functools.partial(
    pl.kernel,
    out_type=jax.ShapeDtypeStruct((2 * NPAD,), _f32),
    mesh=_MESH,
    compiler_params=_SC_PARAMS,
    scratch_types=[
        pltpu.VMEM((64, 128), _i32),
        pltpu.VMEM((128,), _f32),
        pltpu.MemorySpace.VMEM_SHARED((NPAD,), _f32),
        pltpu.SemaphoreType.DMA,
    ],
)
def _deg_kernel(dst2d, zeros1k, out, didx, ones_v, acc_sh, ssem):
    c = lax.axis_index("c")
    s = lax.axis_index("s")
    wid = s * 2 + c
    for i in range(8):
        ones_v[pl.ds(16 * i, 16)] = jnp.ones((16,), _f32)
    for z in range(6):
        pltpu.sync_copy(zeros1k, acc_sh.at[pl.ds(s * RPT + z * 1024, 1024)])
    pltpu.sync_copy(zeros1k.at[pl.ds(0, 128)],
                    acc_sh.at[pl.ds(s * RPT + 6144, 128)])
    plsc.subcore_barrier()

    def group(row0, nrows=8):
        sds = [pltpu.async_copy(ones_v, acc_sh.at[didx.at[row0 + k]],
                                ssem, add=True) for k in range(nrows)]
        for d in sds:
            d.wait()

    def super_body(S, carry):
        pltpu.sync_copy(dst2d.at[pl.ds(wid * MAIN_ROWS + 64 * S, 64)], didx)

        def gbody(g, c2):
            group(16 * g, 16)
            return c2

        lax.fori_loop(0, 4, gbody, 0)
        return carry

    lax.fori_loop(0, SUPERS, super_body, 0)
    pltpu.sync_copy(dst2d.at[pl.ds(wid * MAIN_ROWS + 64 * SUPERS, 8)],
                    didx.at[pl.ds(0, 8)])
    group(0)

    @pl.when(wid < LEFT_MACROS)
    def _():
        pltpu.sync_copy(dst2d.at[pl.ds(LEFT_ROW0 + wid * 8, 8)],
                        didx.at[pl.ds(0, 8)])
        group(0)

    plsc.subcore_barrier()
    pltpu.sync_copy(acc_sh.at[pl.ds(s * RPT, RPT)],
                    out.at[pl.ds(c * NPAD + s * RPT, RPT)])


def _make_prop_kernel(F):
    @functools.partial(
        pl.kernel,
        out_type=jax.ShapeDtypeStruct((2 * NPAD, F), _f32),
        mesh=_MESH,
        compiler_params=_SC_PARAMS,
        scratch_types=[
            pltpu.VMEM((64, 128), _i32),
            pltpu.VMEM((64, 128), _i32),
            pltpu.VMEM((1024, F), _f32),
            pltpu.MemorySpace.VMEM_SHARED((NPAD, F), _f32),
            pltpu.MemorySpace.VMEM_SHARED((NPAD, F), _f32),
            pltpu.SemaphoreType.DMA,
            pltpu.SemaphoreType.DMA,
        ],
    )
    def _prop(u, src2d, dst2d, zrows, out, sidx, didx, rows_v, tab_sh, acc_sh,
              gsem, ssem):
        c = lax.axis_index("c")
        s = lax.axis_index("s")
        wid = s * 2 + c
        pltpu.sync_copy(u.at[pl.ds(s * RPT, RPT)], tab_sh.at[pl.ds(s * RPT, RPT)])
        for z in range(6):
            pltpu.sync_copy(zrows, acc_sh.at[pl.ds(s * RPT + z * 1024, 1024)])
        pltpu.sync_copy(zrows.at[pl.ds(0, 128)],
                        acc_sh.at[pl.ds(s * RPT + 6144, 128)])
        plsc.subcore_barrier()

        def group(row0):
            gds = [pltpu.async_copy(tab_sh.at[sidx.at[row0]],
                                    rows_v.at[pl.ds(0, 128)], gsem)]
            sds = []
            for k in range(8):
                if k < 7:
                    gds.append(pltpu.async_copy(
                        tab_sh.at[sidx.at[row0 + k + 1]],
                        rows_v.at[pl.ds(128 * (k + 1), 128)], gsem))
                gds[k].wait()
                sds.append(pltpu.async_copy(
                    rows_v.at[pl.ds(128 * k, 128)],
                    acc_sh.at[didx.at[row0 + k]], ssem, add=True))
            for d in sds:
                d.wait()

        def super_body(S, carry):
            rb = wid * MAIN_ROWS + 64 * S
            pltpu.sync_copy(src2d.at[pl.ds(rb, 64)], sidx)
            pltpu.sync_copy(dst2d.at[pl.ds(rb, 64)], didx)

            def gbody(g, c2):
                group(8 * g)
                return c2

            lax.fori_loop(0, 8, gbody, 0)
            return carry

        lax.fori_loop(0, SUPERS, super_body, 0)
        rb_tail = wid * MAIN_ROWS + 64 * SUPERS
        pltpu.sync_copy(src2d.at[pl.ds(rb_tail, 8)], sidx.at[pl.ds(0, 8)])
        pltpu.sync_copy(dst2d.at[pl.ds(rb_tail, 8)], didx.at[pl.ds(0, 8)])
        group(0)

        @pl.when(wid < LEFT_MACROS)
        def _():
            pltpu.sync_copy(src2d.at[pl.ds(LEFT_ROW0 + wid * 8, 8)],
                            sidx.at[pl.ds(0, 8)])
            pltpu.sync_copy(dst2d.at[pl.ds(LEFT_ROW0 + wid * 8, 8)],
                            didx.at[pl.ds(0, 8)])
            group(0)

        plsc.subcore_barrier()
        pltpu.sync_copy(acc_sh.at[pl.ds(s * RPT, RPT)],
                        out.at[pl.ds(c * NPAD + s * RPT, RPT)])

    return _prop


_prop8 = _make_prop_kernel(8)


@functools.partial(
    pl.kernel,
    out_type=jax.ShapeDtypeStruct((BPOOL,), _f32),
    mesh=_MESH,
    compiler_params=_SC_PARAMS,
    scratch_types=[
        pltpu.VMEM((49, 128), _i32),
        pltpu.VMEM((49, 128), _f32),
        pltpu.MemorySpace.VMEM_SHARED((BPOOL,), _f32),
    ],
)
def _pool_kernel(v2d, b2d, zeros1k, out, bidx, vv, acc_sh):
    c = lax.axis_index("c")
    s = lax.axis_index("s")

    @pl.when((c == 0) & (s == 0))
    def _():
        pltpu.sync_copy(zeros1k, acc_sh.at[pl.ds(0, 1024)])
        pltpu.sync_copy(zeros1k.at[pl.ds(0, 128)], acc_sh.at[pl.ds(1024, 128)])

    plsc.subcore_barrier()

    @pl.when(c == 0)
    def _():
        pltpu.sync_copy(b2d.at[pl.ds(s * 49, 49)], bidx)
        pltpu.sync_copy(v2d.at[pl.ds(s * 49, 49)], vv)
        for r in range(49):
            pltpu.sync_copy(vv.at[r], acc_sh.at[bidx.at[r]], add=True)

    plsc.subcore_barrier()

    @pl.when((c == 0) & (s == 0))
    def _():
        pltpu.sync_copy(acc_sh, out)


_BN = 112


def _tc_a_body(cnt_ref, xt_ref, dinv_ref, u1t_ref):
    csum = cnt_ref[0] + cnt_ref[1]
    dinv = lax.rsqrt(csum + 1.0)
    dinv_ref[...] = dinv
    for k in range(4):
        u1t_ref[k] = xt_ref[k] * dinv


def _tc_b_body(dinv_ref, u1t_ref, s1t_ref, w1_ref, b1_ref, w2_ref, u2t_ref):
    dinv = dinv_ref[...]
    g1 = [dinv * (s1t_ref[0, k] + s1t_ref[1, k] + u1t_ref[k]) for k in range(4)]
    h1 = []
    for f in range(16):
        acc = jnp.full_like(dinv, b1_ref[f])
        for k in range(4):
            acc = acc + g1[k] * w1_ref[k, f]
        h1.append(jnp.tanh(acc))
    for j in range(8):
        z = h1[0] * w2_ref[0, j]
        for f in range(1, 16):
            z = z + h1[f] * w2_ref[f, j]
        u2t_ref[j] = dinv * z


def _tc_c_body(dinv_ref, u2t_ref, s2t_ref, b2_ref, w4_ref, v_ref):
    dinv = dinv_ref[...]
    v = None
    for j in range(8):
        g = dinv * (s2t_ref[0, j] + s2t_ref[1, j] + u2t_ref[j]) + b2_ref[j]
        t = jnp.tanh(g) * w4_ref[j, 0]
        v = t if v is None else v + t
    v_ref[...] = v


def _tc_d_body(pool_ref, addt_ref, w3_ref, b3_ref, w4_ref, b4_ref, out_ref):
    acc = pool_ref[pl.ds(0, 8), :] + b4_ref[0]
    for f in range(8):
        ax = jnp.full_like(acc, b3_ref[f])
        for k in range(4):
            ax = ax + addt_ref[k] * w3_ref[k, f]
        acc = acc + jnp.tanh(ax) * w4_ref[8 + f, 0]
    out_ref[...] = acc


def _smem_spec(shape):
    return pl.BlockSpec(shape, lambda i: (0,) * len(shape),
                        memory_space=pltpu.SMEM)


_tc_a = pl.pallas_call(
    _tc_a_body,
    grid=(RROWS // _BN,),
    in_specs=[
        pl.BlockSpec((2, _BN, 128), lambda i: (0, i, 0)),
        pl.BlockSpec((4, _BN, 128), lambda i: (0, i, 0)),
    ],
    out_specs=[
        pl.BlockSpec((_BN, 128), lambda i: (i, 0)),
        pl.BlockSpec((4, _BN, 128), lambda i: (0, i, 0)),
    ],
    out_shape=[
        jax.ShapeDtypeStruct((RROWS, 128), _f32),
        jax.ShapeDtypeStruct((4, RROWS, 128), _f32),
    ],
)

_tc_b = pl.pallas_call(
    _tc_b_body,
    grid=(RROWS // _BN,),
    in_specs=[
        pl.BlockSpec((_BN, 128), lambda i: (i, 0)),
        pl.BlockSpec((4, _BN, 128), lambda i: (0, i, 0)),
        pl.BlockSpec((2, 4, _BN, 128), lambda i: (0, 0, i, 0)),
        _smem_spec((4, 16)),
        _smem_spec((16,)),
        _smem_spec((16, 8)),
    ],
    out_specs=[pl.BlockSpec((8, _BN, 128), lambda i: (0, i, 0))],
    out_shape=[jax.ShapeDtypeStruct((8, RROWS, 128), _f32)],
)

_tc_c = pl.pallas_call(
    _tc_c_body,
    grid=(RROWS // _BN,),
    in_specs=[
        pl.BlockSpec((_BN, 128), lambda i: (i, 0)),
        pl.BlockSpec((8, _BN, 128), lambda i: (0, i, 0)),
        pl.BlockSpec((2, 8, _BN, 128), lambda i: (0, 0, i, 0)),
        _smem_spec((8,)),
        _smem_spec((16, 1)),
    ],
    out_specs=[pl.BlockSpec((_BN, 128), lambda i: (i, 0))],
    out_shape=[jax.ShapeDtypeStruct((RROWS, 128), _f32)],
)

_tc_d = pl.pallas_call(
    _tc_d_body,
    grid=(1,),
    in_specs=[
        pl.BlockSpec((9, 128), lambda i: (0, 0)),
        pl.BlockSpec((4, 8, 128), lambda i: (0, 0, 0)),
        _smem_spec((4, 8)),
        _smem_spec((8,)),
        _smem_spec((16, 1)),
        _smem_spec((1,)),
    ],
    out_specs=[pl.BlockSpec((8, 128), lambda i: (0, 0))],
    out_shape=[jax.ShapeDtypeStruct((8, 128), _f32)],
)


def kernel(x, edge_index, batch, y, p, c, apf, wiener,
           W1, b1, W2, b2, W3, b3, W4, b4):
    src2d = edge_index[0].reshape(ER, 128)
    dst2d = edge_index[1].reshape(ER, 128)
    zeros1k = jnp.zeros((1024,), _f32)
    zrows8 = jnp.zeros((1024, 8), _f32)

    cnt = _deg_kernel(dst2d, zeros1k).reshape(2, RROWS, 128)

    xt = jnp.pad(x, ((0, NPAD - NN), (0, 0))).T.reshape(4, RROWS, 128)
    dinv, u1t = _tc_a(cnt, xt)

    u1 = jnp.concatenate(
        [u1t, jnp.zeros((4, RROWS, 128), _f32)]).reshape(8, NPAD).T
    s1 = _prop8(u1, src2d, dst2d, zrows8)
    s1t = (s1.reshape(2, NPAD, 8).transpose(0, 2, 1)[:, :4]
           .reshape(2, 4, RROWS, 128))

    (u2t,) = _tc_b(dinv, u1t, s1t, W1, b1, W2)
    u2 = u2t.reshape(8, NPAD).T
    s2 = _prop8(u2, src2d, dst2d, zrows8)
    s2t = s2.reshape(2, NPAD, 8).transpose(0, 2, 1).reshape(2, 8, RROWS, 128)

    (v,) = _tc_c(dinv, u2t, s2t, b2, W4)

    padids = (BB + (jnp.arange(NPAD - NN, dtype=_i32) % 128)).astype(_i32)
    b2d = jnp.concatenate([batch, padids]).reshape(RROWS, 128)
    pool = _pool_kernel(v, b2d, zeros1k)

    addt = jnp.stack([p, c, apf, wiener]).reshape(4, 8, 128)
    (out2d,) = _tc_d(pool.reshape(9, 128), addt, W3, b3, W4, b4)
    return out2d.reshape(BB, 1)
